# Initial kernel scaffold; baseline (speedup 1.0000x reference)
import jax
import jax.numpy as jnp
from jax.experimental import pallas as pl


def kernel(z, cb):
    raise NotImplementedError("write your pallas kernel here")


# speedup vs baseline: 16.9489x; 16.9489x over previous
import functools

import jax
import jax.numpy as jnp
from jax.experimental import pallas as pl


def _vq_kernel(z_ref, cb_ref, zq_ref, idx_ref, vq_ref):
    n = pl.program_id(0)
    b = pl.program_id(1)

    zb = z_ref[0]
    cbn = cb_ref[0]

    xc = jax.lax.dot_general(
        cbn, zb, (((1,), (0,)), ((), ())),
        preferred_element_type=jnp.float32)
    c2 = jnp.sum(cbn * cbn, axis=1, keepdims=True)
    x2 = jnp.sum(zb * zb, axis=0, keepdims=True)
    scores = jnp.maximum(x2 + c2 - 2.0 * xc, 0.0)

    idx = jnp.argmin(scores, axis=0)

    onehot = (jax.lax.broadcasted_iota(jnp.int32, scores.shape, 0)
              == idx[None, :]).astype(jnp.float32)
    qT = jax.lax.dot_general(
        cbn, onehot, (((0,), (0,)), ((), ())),
        preferred_element_type=jnp.float32)

    zq_ref[0] = qT
    idx_ref[0, 0, 0] = idx

    @pl.when(jnp.logical_and(n == 0, b == 0))
    def _():
        vq_ref[...] = jnp.zeros((1, 1), jnp.float32)

    diff = zb - qT
    vq_ref[...] += jnp.sum(diff * diff).reshape(1, 1)


@functools.partial(jax.jit, static_argnames=())
def kernel(z, cb):
    beta = 0.25
    B, D, H, W = z.shape
    N, M, ds = cb.shape
    T = B * H * W
    HW = H * W

    zr = z.reshape(B, D, HW)

    zq, idx, vq = pl.pallas_call(
        _vq_kernel,
        grid=(N, B),
        in_specs=[
            pl.BlockSpec((1, ds, HW), lambda n, b: (b, n, 0)),
            pl.BlockSpec((1, M, ds), lambda n, b: (n, 0, 0)),
        ],
        out_specs=[
            pl.BlockSpec((1, ds, HW), lambda n, b: (b, n, 0)),
            pl.BlockSpec((1, 1, 1, HW), lambda n, b: (b, n, 0, 0)),
            pl.BlockSpec((1, 1), lambda n, b: (0, 0)),
        ],
        out_shape=[
            jax.ShapeDtypeStruct((B, D, HW), jnp.float32),
            jax.ShapeDtypeStruct((B, N, 1, HW), jnp.int32),
            jax.ShapeDtypeStruct((1, 1), jnp.float32),
        ],
    )(zr, cb)

    z_q = zq.reshape(B, D, H, W)
    indices = idx.reshape(B, N, H, W)
    vq_loss = (1.0 + beta) * vq[0, 0] / (N * T * ds)
    return (z_q, vq_loss, indices)


# speedup vs baseline: 18.9365x; 1.1173x over previous
import functools

import jax
import jax.numpy as jnp
from jax.experimental import pallas as pl


def _vq_kernel(z_ref, cb_ref, cba_ref, zq_ref, idx_ref, vq_ref):
    n = pl.program_id(0)
    b = pl.program_id(1)

    zb = z_ref[0]
    cbn = cb_ref[0]
    cba = cba_ref[0]

    x2 = jnp.sum(zb * zb, axis=0, keepdims=True)
    ones = jnp.ones_like(x2)
    z_aug = jnp.concatenate([-2.0 * zb, ones, x2], axis=0)
    scores = jax.lax.dot_general(
        cba, z_aug, (((1,), (0,)), ((), ())),
        preferred_element_type=jnp.float32)

    idx = jnp.argmin(scores, axis=0)

    onehot = (jax.lax.broadcasted_iota(jnp.int32, scores.shape, 0)
              == idx[None, :]).astype(jnp.float32)
    qT = jax.lax.dot_general(
        cbn, onehot, (((0,), (0,)), ((), ())),
        preferred_element_type=jnp.float32)

    zq_ref[0] = qT
    idx_ref[0, 0, 0] = idx

    @pl.when(jnp.logical_and(n == 0, b == 0))
    def _():
        vq_ref[...] = jnp.zeros((1, 1), jnp.float32)

    diff = zb - qT
    vq_ref[...] += jnp.sum(diff * diff).reshape(1, 1)


@functools.partial(jax.jit, static_argnames=())
def kernel(z, cb):
    beta = 0.25
    B, D, H, W = z.shape
    N, M, ds = cb.shape
    T = B * H * W
    HW = H * W

    zr = z.reshape(B, D, HW)
    c2 = jnp.sum(cb * cb, axis=2, keepdims=True)
    cb_aug = jnp.concatenate(
        [cb, c2, jnp.ones_like(c2)], axis=2)

    zq, idx, vq = pl.pallas_call(
        _vq_kernel,
        grid=(N, B),
        in_specs=[
            pl.BlockSpec((1, ds, HW), lambda n, b: (b, n, 0)),
            pl.BlockSpec((1, M, ds), lambda n, b: (n, 0, 0)),
            pl.BlockSpec((1, M, ds + 2), lambda n, b: (n, 0, 0)),
        ],
        out_specs=[
            pl.BlockSpec((1, ds, HW), lambda n, b: (b, n, 0)),
            pl.BlockSpec((1, 1, 1, HW), lambda n, b: (b, n, 0, 0)),
            pl.BlockSpec((1, 1), lambda n, b: (0, 0)),
        ],
        out_shape=[
            jax.ShapeDtypeStruct((B, D, HW), jnp.float32),
            jax.ShapeDtypeStruct((B, N, 1, HW), jnp.int32),
            jax.ShapeDtypeStruct((1, 1), jnp.float32),
        ],
    )(zr, cb, cb_aug)

    z_q = zq.reshape(B, D, H, W)
    indices = idx.reshape(B, N, H, W)
    vq_loss = (1.0 + beta) * vq[0, 0] / (N * T * ds)
    return (z_q, vq_loss, indices)
